# Initial kernel scaffold; baseline (speedup 1.0000x reference)
import jax
import jax.numpy as jnp
from jax.experimental import pallas as pl


def kernel(x, w1, bn1, w2, bn2, geo_w, geo_bn, feat_w, feat_bn, fuse_w, fuse_bn, gl0_w, gl0_bn, gl1_w, gl1_bn, wf, bnf):
    raise NotImplementedError("write your pallas kernel here")


# speedup vs baseline: 1.0090x; 1.0090x over previous
import functools
import jax
import jax.numpy as jnp
import numpy as np
from jax.experimental import pallas as pl
from jax.experimental.pallas import tpu as pltpu

_EPS = 1e-5


def _fold_bn(w, bnp):
    g, b, m, v = bnp
    s = g * jax.lax.rsqrt(v + _EPS)
    return w * s[:, None], b - m * s


def _knn(query, ref, k):
    d = (jnp.sum(query * query, -1)[:, :, None]
         - 2.0 * jnp.einsum('bsd,bnd->bsn', query, ref)
         + jnp.sum(ref * ref, -1)[:, None, :])
    return jax.lax.top_k(-d, k)[1]


def _fps(xyz, npoint):
    B, N, _ = xyz.shape
    def body(i, state):
        idxs, dists, far = state
        idxs = idxs.at[:, i].set(far)
        centroid = jnp.take_along_axis(xyz, far[:, None, None], axis=1)
        d = jnp.sum((xyz - centroid) ** 2, -1)
        dists = jnp.minimum(dists, d)
        return idxs, dists, jnp.argmax(dists, -1).astype(jnp.int32)
    idxs = jnp.zeros((B, npoint), jnp.int32)
    dists = jnp.full((B, N), 1e10, xyz.dtype)
    far = jnp.zeros((B,), jnp.int32)
    idxs, _, _ = jax.lax.fori_loop(0, npoint, body, (idxs, dists, far))
    return idxs


def _sample_and_group(npoint, nsample, xyz, feature):
    B = xyz.shape[0]
    fps_idx = _fps(xyz, npoint)
    bi = jnp.arange(B)[:, None]
    new_xyz = xyz[bi, fps_idx]
    ft = jnp.swapaxes(feature, 1, 2)
    new_pts = ft[bi, fps_idx]
    idx = _knn(new_xyz, xyz, nsample)
    grouped = ft[jnp.arange(B)[:, None, None], idx]
    centered = grouped - new_pts[:, :, None, :]
    new_feature = jnp.concatenate(
        [centered, jnp.broadcast_to(new_pts[:, :, None, :], grouped.shape)], -1)
    return new_xyz, new_feature


def _local_op(x, w, bnp):
    wp, bp = _fold_bn(w, bnp)
    h = jax.nn.relu(jnp.einsum('bskd,od->bsko', x, wp) + bp)
    return jnp.swapaxes(jnp.max(h, axis=2), 1, 2)


def _edge_branch(f_t, idx, w, bnp):
    B = f_t.shape[0]
    wp, bp = _fold_bn(w, bnp)
    nb = f_t[jnp.arange(B)[:, None, None], idx]
    center = jnp.broadcast_to(f_t[:, :, None, :], nb.shape)
    e = jnp.concatenate([nb - center, center], -1)
    h = jax.nn.leaky_relu(jnp.einsum('bnkc,oc->bnko', e, wp) + bp, 0.2)
    return jnp.max(h, axis=2)


def _dual_graph_fusion(xyz_cf, f, k, geo_w, geo_bn, feat_w, feat_bn, fuse_w, fuse_bn):
    xyz_t = jnp.swapaxes(xyz_cf, 1, 2)
    f_t = jnp.swapaxes(f, 1, 2)
    idx_g = _knn(xyz_t, xyz_t, k)
    idx_f = _knn(f_t, f_t, k)
    g = _edge_branch(f_t, idx_g, geo_w, geo_bn)
    h = _edge_branch(f_t, idx_f, feat_w, feat_bn)
    fused = jnp.concatenate([g, h], -1)
    wp, bp = _fold_bn(fuse_w, fuse_bn)
    out = jax.nn.leaky_relu(jnp.einsum('bnc,oc->bno', fused, wp) + bp, 0.2)
    return jnp.swapaxes(out, 1, 2)


def _final_kernel(f1_ref, xs_ref, w2_ref, b2_ref, wfa_ref, wfb_ref, bf_ref, o_ref):
    xs = xs_ref[0]
    f1 = f1_ref[0]
    xskip = jnp.maximum(
        jnp.dot(w2_ref[...], xs, preferred_element_type=jnp.float32)
        + b2_ref[...][:, None], 0.0)
    acc = (jnp.dot(wfa_ref[...], f1, preferred_element_type=jnp.float32)
           + jnp.dot(wfb_ref[...], xskip, preferred_element_type=jnp.float32)
           + bf_ref[...][:, None])
    o_ref[0] = jnp.where(acc >= 0.0, acc, 0.2 * acc)


def _final_stage(f1, x_strided, w2, bn2, wf, bnf):
    B = f1.shape[0]
    w2p, b2p = _fold_bn(w2, bn2)
    wfp, bfp = _fold_bn(wf, bnf)
    wfa = wfp[:, :256]
    wfb = wfp[:, 256:]
    return pl.pallas_call(
        _final_kernel,
        grid=(B,),
        in_specs=[
            pl.BlockSpec((1, 256, 256), lambda b: (b, 0, 0)),
            pl.BlockSpec((1, 64, 256), lambda b: (b, 0, 0)),
            pl.BlockSpec((1024, 64), lambda b: (0, 0)),
            pl.BlockSpec((1024,), lambda b: (0,)),
            pl.BlockSpec((512, 256), lambda b: (0, 0)),
            pl.BlockSpec((512, 1024), lambda b: (0, 0)),
            pl.BlockSpec((512,), lambda b: (0,)),
        ],
        out_specs=pl.BlockSpec((1, 512, 256), lambda b: (b, 0, 0)),
        out_shape=jax.ShapeDtypeStruct((B, 512, 256), jnp.float32),
        compiler_params=pltpu.CompilerParams(
            dimension_semantics=("parallel",)),
    )(f1, x_strided, w2p, b2p, wfa, wfb, bfp)


@jax.jit
def kernel(x, w1, bn1, w2, bn2, geo_w, geo_bn, feat_w, feat_bn, fuse_w, fuse_bn,
           gl0_w, gl0_bn, gl1_w, gl1_bn, wf, bnf):
    B, _, N = x.shape
    xyz = jnp.swapaxes(x[:, 0:3, :], 1, 2)
    g1, be1, m1, v1 = bn1
    h1 = jnp.einsum('bcn,oc->bon', x, w1)
    f_o = jax.nn.relu((h1 - m1[:, None]) * g1[:, None]
                      * jax.lax.rsqrt(v1[:, None] + _EPS) + be1[:, None])
    out_feat = _dual_graph_fusion(x[:, 0:3, :], f_o, 20,
                                  geo_w, geo_bn, feat_w, feat_bn, fuse_w, fuse_bn)
    stride = N // 256
    new_xyz, nf = _sample_and_group(512, 32, xyz, out_feat)
    f0 = _local_op(nf, gl0_w, gl0_bn)
    new_xyz, nf = _sample_and_group(256, 32, new_xyz, f0)
    f1 = _local_op(nf, gl1_w, gl1_bn)
    return _final_stage(f1, f_o[:, :, ::stride], w2, bn2, wf, bnf)


# speedup vs baseline: 1.1326x; 1.1224x over previous
import functools
import jax
import jax.numpy as jnp
import numpy as np
from jax.experimental import pallas as pl
from jax.experimental.pallas import tpu as pltpu

_EPS = 1e-5


def _fold_bn(w, bnp):
    g, b, m, v = bnp
    s = g * jax.lax.rsqrt(v + _EPS)
    return w * s[:, None], b - m * s


def _knn(query, ref, k):
    d = (jnp.sum(query * query, -1)[:, :, None]
         - 2.0 * jnp.einsum('bsd,bnd->bsn', query, ref)
         + jnp.sum(ref * ref, -1)[:, None, :])
    return jax.lax.top_k(-d, k)[1]


def _fps_kernel(xr, yr, zr, idx_ref, nxr, nyr, nzr, *, npoint, n):
    X = xr[...]; Y = yr[...]; Z = zr[...]
    rb = X.shape[0]
    li = jax.lax.broadcasted_iota(jnp.int32, (rb, n), 1)
    si = jax.lax.broadcasted_iota(jnp.int32, (rb, npoint), 1)

    def bcast_n(v):
        return jnp.broadcast_to(v, (rb, n))

    def bcast_s(v):
        return jnp.broadcast_to(v, (rb, npoint))

    def body(i, carry):
        idxs, cxs, cys, czs, dists, far = carry
        sel = si == i
        idxs = jnp.where(sel, bcast_s(far), idxs)
        mask = li == bcast_n(far)
        cx = jnp.sum(jnp.where(mask, X, 0.0), axis=1, keepdims=True)
        cy = jnp.sum(jnp.where(mask, Y, 0.0), axis=1, keepdims=True)
        cz = jnp.sum(jnp.where(mask, Z, 0.0), axis=1, keepdims=True)
        cxs = jnp.where(sel, bcast_s(cx), cxs)
        cys = jnp.where(sel, bcast_s(cy), cys)
        czs = jnp.where(sel, bcast_s(cz), czs)
        dx = X - bcast_n(cx); dy = Y - bcast_n(cy); dz = Z - bcast_n(cz)
        d = dx * dx + dy * dy + dz * dz
        dists = jnp.minimum(dists, d)
        m = jnp.max(dists, axis=1, keepdims=True)
        far = jnp.min(jnp.where(dists == bcast_n(m), li, n),
                      axis=1, keepdims=True)
        return idxs, cxs, cys, czs, dists, far

    zs_s = X[:, :npoint] * 0.0
    init = (zs_s.astype(jnp.int32),
            zs_s, zs_s, zs_s,
            X * 0.0 + 1e10,
            (X[:, :1] * 0.0).astype(jnp.int32))
    idxs, cxs, cys, czs, _, _ = jax.lax.fori_loop(0, npoint, body, init)
    idx_ref[...] = idxs
    nxr[...] = cxs; nyr[...] = cys; nzr[...] = czs


def _fps(xs, ys, zs, npoint):
    B, N = xs.shape
    rb = B // 2
    spec_in = pl.BlockSpec((rb, N), lambda b: (b, 0))
    spec_out = pl.BlockSpec((rb, npoint), lambda b: (b, 0))
    return pl.pallas_call(
        functools.partial(_fps_kernel, npoint=npoint, n=N),
        grid=(2,),
        in_specs=[spec_in] * 3,
        out_specs=[spec_out] * 4,
        out_shape=[jax.ShapeDtypeStruct((B, npoint), jnp.int32)]
        + [jax.ShapeDtypeStruct((B, npoint), jnp.float32)] * 3,
        compiler_params=pltpu.CompilerParams(
            dimension_semantics=("parallel",)),
    )(xs, ys, zs)


def _sample_and_group(npoint, nsample, xs, ys, zs, xyz, feature):
    B = xyz.shape[0]
    fps_idx, nx, ny, nz = _fps(xs, ys, zs, npoint)
    bi = jnp.arange(B)[:, None]
    new_xyz = jnp.stack([nx, ny, nz], axis=-1)
    ft = jnp.swapaxes(feature, 1, 2)
    new_pts = ft[bi, fps_idx]
    idx = _knn(new_xyz, xyz, nsample)
    grouped = ft[jnp.arange(B)[:, None, None], idx]
    centered = grouped - new_pts[:, :, None, :]
    new_feature = jnp.concatenate(
        [centered, jnp.broadcast_to(new_pts[:, :, None, :], grouped.shape)], -1)
    return (nx, ny, nz), new_xyz, new_feature


def _local_op(x, w, bnp):
    wp, bp = _fold_bn(w, bnp)
    h = jax.nn.relu(jnp.einsum('bskd,od->bsko', x, wp) + bp)
    return jnp.swapaxes(jnp.max(h, axis=2), 1, 2)


def _edge_branch(f_t, idx, w, bnp):
    B = f_t.shape[0]
    wp, bp = _fold_bn(w, bnp)
    nb = f_t[jnp.arange(B)[:, None, None], idx]
    center = jnp.broadcast_to(f_t[:, :, None, :], nb.shape)
    e = jnp.concatenate([nb - center, center], -1)
    h = jax.nn.leaky_relu(jnp.einsum('bnkc,oc->bnko', e, wp) + bp, 0.2)
    return jnp.max(h, axis=2)


def _dual_graph_fusion(xyz_cf, f, k, geo_w, geo_bn, feat_w, feat_bn, fuse_w, fuse_bn):
    xyz_t = jnp.swapaxes(xyz_cf, 1, 2)
    f_t = jnp.swapaxes(f, 1, 2)
    idx_g = _knn(xyz_t, xyz_t, k)
    idx_f = _knn(f_t, f_t, k)
    g = _edge_branch(f_t, idx_g, geo_w, geo_bn)
    h = _edge_branch(f_t, idx_f, feat_w, feat_bn)
    fused = jnp.concatenate([g, h], -1)
    wp, bp = _fold_bn(fuse_w, fuse_bn)
    out = jax.nn.leaky_relu(jnp.einsum('bnc,oc->bno', fused, wp) + bp, 0.2)
    return jnp.swapaxes(out, 1, 2)


def _final_kernel(f1_ref, xs_ref, w2_ref, b2_ref, wfa_ref, wfb_ref, bf_ref, o_ref):
    xs = xs_ref[0]
    f1 = f1_ref[0]
    xskip = jnp.maximum(
        jnp.dot(w2_ref[...], xs, preferred_element_type=jnp.float32)
        + b2_ref[...][:, None], 0.0)
    acc = (jnp.dot(wfa_ref[...], f1, preferred_element_type=jnp.float32)
           + jnp.dot(wfb_ref[...], xskip, preferred_element_type=jnp.float32)
           + bf_ref[...][:, None])
    o_ref[0] = jnp.where(acc >= 0.0, acc, 0.2 * acc)


def _final_stage(f1, x_strided, w2, bn2, wf, bnf):
    B = f1.shape[0]
    w2p, b2p = _fold_bn(w2, bn2)
    wfp, bfp = _fold_bn(wf, bnf)
    wfa = wfp[:, :256]
    wfb = wfp[:, 256:]
    return pl.pallas_call(
        _final_kernel,
        grid=(B,),
        in_specs=[
            pl.BlockSpec((1, 256, 256), lambda b: (b, 0, 0)),
            pl.BlockSpec((1, 64, 256), lambda b: (b, 0, 0)),
            pl.BlockSpec((1024, 64), lambda b: (0, 0)),
            pl.BlockSpec((1024,), lambda b: (0,)),
            pl.BlockSpec((512, 256), lambda b: (0, 0)),
            pl.BlockSpec((512, 1024), lambda b: (0, 0)),
            pl.BlockSpec((512,), lambda b: (0,)),
        ],
        out_specs=pl.BlockSpec((1, 512, 256), lambda b: (b, 0, 0)),
        out_shape=jax.ShapeDtypeStruct((B, 512, 256), jnp.float32),
        compiler_params=pltpu.CompilerParams(
            dimension_semantics=("parallel",)),
    )(f1, x_strided, w2p, b2p, wfa, wfb, bfp)


@jax.jit
def kernel(x, w1, bn1, w2, bn2, geo_w, geo_bn, feat_w, feat_bn, fuse_w, fuse_bn,
           gl0_w, gl0_bn, gl1_w, gl1_bn, wf, bnf):
    B, _, N = x.shape
    xyz = jnp.swapaxes(x[:, 0:3, :], 1, 2)
    g1, be1, m1, v1 = bn1
    h1 = jnp.einsum('bcn,oc->bon', x, w1)
    f_o = jax.nn.relu((h1 - m1[:, None]) * g1[:, None]
                      * jax.lax.rsqrt(v1[:, None] + _EPS) + be1[:, None])
    out_feat = _dual_graph_fusion(x[:, 0:3, :], f_o, 20,
                                  geo_w, geo_bn, feat_w, feat_bn, fuse_w, fuse_bn)
    stride = N // 256
    xs = x[:, 0, :]; ys = x[:, 1, :]; zs = x[:, 2, :]
    (nx, ny, nz), new_xyz, nf = _sample_and_group(512, 32, xs, ys, zs, xyz, out_feat)
    f0 = _local_op(nf, gl0_w, gl0_bn)
    _, new_xyz, nf = _sample_and_group(256, 32, nx, ny, nz, new_xyz, f0)
    f1 = _local_op(nf, gl1_w, gl1_bn)
    return _final_stage(f1, f_o[:, :, ::stride], w2, bn2, wf, bnf)


# speedup vs baseline: 7.7134x; 6.8104x over previous
import functools
import jax
import jax.numpy as jnp
import numpy as np
from jax.experimental import pallas as pl
from jax.experimental.pallas import tpu as pltpu

_EPS = 1e-5


def _fold_bn(w, bnp):
    g, b, m, v = bnp
    s = g * jax.lax.rsqrt(v + _EPS)
    return w * s[:, None], b - m * s


def _knn_agg_kernel(q_ref, r_ref, qq_ref, rr_ref, v_ref, t_ref, o_ref, *, k, n, slope):
    q = q_ref[0]
    r = r_ref[0]
    v = v_ref[0]
    sb = q.shape[0]
    c = v.shape[1]
    qq = qq_ref[0]
    rr = rr_ref[0]
    qr = jax.lax.dot_general(q, r, (((1,), (1,)), ((), ())),
                             preferred_element_type=jnp.float32)
    d2 = (jnp.broadcast_to(qq, (sb, n)) - 2.0 * qr) + jnp.broadcast_to(rr, (sb, n))
    li = jax.lax.broadcasted_iota(jnp.int32, (sb, n), 1)

    def body(_, carry):
        d2, out = carry
        m = jnp.min(d2, axis=1, keepdims=True)
        eqm = d2 == jnp.broadcast_to(m, (sb, n))
        mi = jnp.min(jnp.where(eqm, li, n), axis=1, keepdims=True)
        oh = li == jnp.broadcast_to(mi, (sb, n))
        d2 = jnp.where(oh, 3.4e38, d2)
        contrib = jax.lax.dot_general(
            oh.astype(jnp.float32), v, (((1,), (0,)), ((), ())),
            preferred_element_type=jnp.float32)
        return d2, jnp.maximum(out, contrib)

    out0 = qr[:, :c] * 0.0 - 3e38
    _, mx = jax.lax.fori_loop(0, k, body, (d2, out0))
    z = mx + t_ref[0]
    o_ref[0] = jnp.where(z >= 0.0, z, slope * z)


def _knn_agg(q, r, v, t, k, slope, sb):
    B, S, D = q.shape
    N = r.shape[1]
    C = v.shape[2]
    qq = jnp.sum(q * q, -1)[:, :, None]
    rr = jnp.sum(r * r, -1)[:, None, :]
    return pl.pallas_call(
        functools.partial(_knn_agg_kernel, k=k, n=N, slope=slope),
        grid=(B, S // sb),
        in_specs=[
            pl.BlockSpec((1, sb, D), lambda b, s: (b, s, 0)),
            pl.BlockSpec((1, N, D), lambda b, s: (b, 0, 0)),
            pl.BlockSpec((1, sb, 1), lambda b, s: (b, s, 0)),
            pl.BlockSpec((1, 1, N), lambda b, s: (b, 0, 0)),
            pl.BlockSpec((1, N, C), lambda b, s: (b, 0, 0)),
            pl.BlockSpec((1, sb, C), lambda b, s: (b, s, 0)),
        ],
        out_specs=pl.BlockSpec((1, sb, C), lambda b, s: (b, s, 0)),
        out_shape=jax.ShapeDtypeStruct((B, S, C), jnp.float32),
        compiler_params=pltpu.CompilerParams(
            dimension_semantics=("parallel", "arbitrary")),
    )(q, r, qq, rr, v, t)


def _fps_kernel(xr, yr, zr, idx_ref, nxr, nyr, nzr, *, npoint, n):
    X = xr[...]; Y = yr[...]; Z = zr[...]
    rb = X.shape[0]
    li = jax.lax.broadcasted_iota(jnp.int32, (rb, n), 1)
    si = jax.lax.broadcasted_iota(jnp.int32, (rb, npoint), 1)

    def bcast_n(v):
        return jnp.broadcast_to(v, (rb, n))

    def bcast_s(v):
        return jnp.broadcast_to(v, (rb, npoint))

    def body(i, carry):
        idxs, cxs, cys, czs, dists, far = carry
        sel = si == i
        idxs = jnp.where(sel, bcast_s(far), idxs)
        mask = li == bcast_n(far)
        cx = jnp.sum(jnp.where(mask, X, 0.0), axis=1, keepdims=True)
        cy = jnp.sum(jnp.where(mask, Y, 0.0), axis=1, keepdims=True)
        cz = jnp.sum(jnp.where(mask, Z, 0.0), axis=1, keepdims=True)
        cxs = jnp.where(sel, bcast_s(cx), cxs)
        cys = jnp.where(sel, bcast_s(cy), cys)
        czs = jnp.where(sel, bcast_s(cz), czs)
        dx = X - bcast_n(cx); dy = Y - bcast_n(cy); dz = Z - bcast_n(cz)
        d = dx * dx + dy * dy + dz * dz
        dists = jnp.minimum(dists, d)
        m = jnp.max(dists, axis=1, keepdims=True)
        far = jnp.min(jnp.where(dists == bcast_n(m), li, n),
                      axis=1, keepdims=True)
        return idxs, cxs, cys, czs, dists, far

    zs_s = X[:, :npoint] * 0.0
    init = (zs_s.astype(jnp.int32),
            zs_s, zs_s, zs_s,
            X * 0.0 + 1e10,
            (X[:, :1] * 0.0).astype(jnp.int32))
    idxs, cxs, cys, czs, _, _ = jax.lax.fori_loop(0, npoint, body, init)
    idx_ref[...] = idxs
    nxr[...] = cxs; nyr[...] = cys; nzr[...] = czs


def _fps(xs, ys, zs, npoint):
    B, N = xs.shape
    rb = B // 2
    spec_in = pl.BlockSpec((rb, N), lambda b: (b, 0))
    spec_out = pl.BlockSpec((rb, npoint), lambda b: (b, 0))
    return pl.pallas_call(
        functools.partial(_fps_kernel, npoint=npoint, n=N),
        grid=(2,),
        in_specs=[spec_in] * 3,
        out_specs=[spec_out] * 4,
        out_shape=[jax.ShapeDtypeStruct((B, npoint), jnp.int32)]
        + [jax.ShapeDtypeStruct((B, npoint), jnp.float32)] * 3,
        compiler_params=pltpu.CompilerParams(
            dimension_semantics=("parallel",)),
    )(xs, ys, zs)


def _knn_agg_xla(q, r, v, t, k, slope, sb):
    d = (jnp.sum(q * q, -1)[:, :, None]
         - 2.0 * jnp.einsum('bsd,bnd->bsn', q, r)
         + jnp.sum(r * r, -1)[:, None, :])
    idx = jax.lax.top_k(-d, k)[1]
    B = q.shape[0]
    nb = v[jnp.arange(B)[:, None, None], idx]
    mx = jnp.max(nb, axis=2) + t
    return jnp.where(mx >= 0, mx, slope * mx)


def _edge_terms(f_t, w, bnp):
    wp, bp = _fold_bn(w, bnp)
    cin = f_t.shape[2]
    wd = wp[:, :cin]
    wcd = wp[:, cin:] - wd
    p = jnp.einsum('bnc,oc->bno', f_t, wd)
    t = jnp.einsum('bnc,oc->bno', f_t, wcd) + bp
    return p, t


def _final_kernel(f1_ref, xs_ref, w2_ref, b2_ref, wfa_ref, wfb_ref, bf_ref, o_ref):
    xs = xs_ref[0]
    f1 = f1_ref[0]
    xskip = jnp.maximum(
        jnp.dot(w2_ref[...], xs, preferred_element_type=jnp.float32)
        + b2_ref[...][:, None], 0.0)
    acc = (jnp.dot(wfa_ref[...], f1, preferred_element_type=jnp.float32)
           + jnp.dot(wfb_ref[...], xskip, preferred_element_type=jnp.float32)
           + bf_ref[...][:, None])
    o_ref[0] = jnp.where(acc >= 0.0, acc, 0.2 * acc)


def _final_stage(f1, x_strided, w2, bn2, wf, bnf):
    B = f1.shape[0]
    w2p, b2p = _fold_bn(w2, bn2)
    wfp, bfp = _fold_bn(wf, bnf)
    wfa = wfp[:, :256]
    wfb = wfp[:, 256:]
    return pl.pallas_call(
        _final_kernel,
        grid=(B,),
        in_specs=[
            pl.BlockSpec((1, 256, 256), lambda b: (b, 0, 0)),
            pl.BlockSpec((1, 64, 256), lambda b: (b, 0, 0)),
            pl.BlockSpec((1024, 64), lambda b: (0, 0)),
            pl.BlockSpec((1024,), lambda b: (0,)),
            pl.BlockSpec((512, 256), lambda b: (0, 0)),
            pl.BlockSpec((512, 1024), lambda b: (0, 0)),
            pl.BlockSpec((512,), lambda b: (0,)),
        ],
        out_specs=pl.BlockSpec((1, 512, 256), lambda b: (b, 0, 0)),
        out_shape=jax.ShapeDtypeStruct((B, 512, 256), jnp.float32),
        compiler_params=pltpu.CompilerParams(
            dimension_semantics=("parallel",)),
    )(f1, x_strided, w2p, b2p, wfa, wfb, bfp)


@jax.jit
def kernel(x, w1, bn1, w2, bn2, geo_w, geo_bn, feat_w, feat_bn, fuse_w, fuse_bn,
           gl0_w, gl0_bn, gl1_w, gl1_bn, wf, bnf):
    B, _, N = x.shape
    xyz = jnp.swapaxes(x[:, 0:3, :], 1, 2)
    g1, be1, m1, v1 = bn1
    h1 = jnp.einsum('bcn,oc->bon', x, w1)
    f_o = jax.nn.relu((h1 - m1[:, None]) * g1[:, None]
                      * jax.lax.rsqrt(v1[:, None] + _EPS) + be1[:, None])
    f_t = jnp.swapaxes(f_o, 1, 2)
    bi = jnp.arange(B)[:, None]

    pg, tg = _edge_terms(f_t, geo_w, geo_bn)
    pf, tf = _edge_terms(f_t, feat_w, feat_bn)
    g = _knn_agg(xyz, xyz, pg, tg, k=20, slope=0.2, sb=512)
    h = _knn_agg(f_t, f_t, pf, tf, k=20, slope=0.2, sb=512)
    wpu, bpu = _fold_bn(fuse_w, fuse_bn)
    fused = jnp.concatenate([g, h], -1)
    out_ft = jax.nn.leaky_relu(jnp.einsum('bnc,oc->bno', fused, wpu) + bpu, 0.2)

    xs = x[:, 0, :]; ys = x[:, 1, :]; zs = x[:, 2, :]
    fps_idx, nx, ny, nz = _fps(xs, ys, zs, 512)
    new_xyz = jnp.stack([nx, ny, nz], axis=-1)
    wp0, bp0 = _fold_bn(gl0_w, gl0_bn)
    wa0 = wp0[:, :64]
    wb0 = wp0[:, 64:] - wa0
    a0 = jnp.einsum('bnc,oc->bno', out_ft, wa0)
    t0 = jnp.einsum('bnc,oc->bno', out_ft[bi, fps_idx], wb0) + bp0
    f0_t = _knn_agg(new_xyz, xyz, a0, t0, k=32, slope=0.0, sb=512)

    idx2, n2x, n2y, n2z = _fps(nx, ny, nz, 256)
    new_xyz2 = jnp.stack([n2x, n2y, n2z], axis=-1)
    wp1, bp1 = _fold_bn(gl1_w, gl1_bn)
    wa1 = wp1[:, :128]
    wb1 = wp1[:, 128:] - wa1
    a1 = jnp.einsum('bnc,oc->bno', f0_t, wa1)
    t1 = jnp.einsum('bnc,oc->bno', f0_t[bi, idx2], wb1) + bp1
    f1_t = _knn_agg(new_xyz2, new_xyz, a1, t1, k=32, slope=0.0, sb=256)

    f1 = jnp.swapaxes(f1_t, 1, 2)
    stride = N // 256
    return _final_stage(f1, f_o[:, :, ::stride], w2, bn2, wf, bnf)


# speedup vs baseline: 7.8793x; 1.0215x over previous
import functools
import jax
import jax.numpy as jnp
import numpy as np
from jax.experimental import pallas as pl
from jax.experimental.pallas import tpu as pltpu

_EPS = 1e-5


def _fold_bn(w, bnp):
    g, b, m, v = bnp
    s = g * jax.lax.rsqrt(v + _EPS)
    return w * s[:, None], b - m * s


def _tb(a, b):
    return jax.lax.dot_general(a, b, (((1,), (1,)), ((), ())),
                               preferred_element_type=jnp.float32)


def _knn_agg_kernel(q_ref, r_ref, qq_ref, rr_ref, vs_ref, wv_ref, ts_ref,
                    wt_ref, b_ref, o_ref, *, k, n, slope):
    q = q_ref[0]
    r = r_ref[0]
    v = _tb(vs_ref[0], wv_ref[...])
    tq = _tb(ts_ref[0], wt_ref[...]) + jnp.broadcast_to(
        b_ref[...], (ts_ref.shape[1], wt_ref.shape[0]))
    sb = q.shape[0]
    c = v.shape[1]
    qq = qq_ref[0]
    rr = rr_ref[0]
    qr = jax.lax.dot_general(q, r, (((1,), (1,)), ((), ())),
                             preferred_element_type=jnp.float32)
    d2 = (jnp.broadcast_to(qq, (sb, n)) - 2.0 * qr) + jnp.broadcast_to(rr, (sb, n))
    li = jax.lax.broadcasted_iota(jnp.int32, (sb, n), 1)

    def body(_, carry):
        d2, out = carry
        m = jnp.min(d2, axis=1, keepdims=True)
        eqm = d2 == jnp.broadcast_to(m, (sb, n))
        mi = jnp.min(jnp.where(eqm, li, n), axis=1, keepdims=True)
        oh = li == jnp.broadcast_to(mi, (sb, n))
        d2 = jnp.where(oh, 3.4e38, d2)
        contrib = jax.lax.dot_general(
            oh.astype(jnp.float32), v, (((1,), (0,)), ((), ())),
            preferred_element_type=jnp.float32)
        return d2, jnp.maximum(out, contrib)

    out0 = qr[:, :c] * 0.0 - 3e38
    _, mx = jax.lax.fori_loop(0, k, body, (d2, out0))
    z = mx + tq
    o_ref[0] = jnp.where(z >= 0.0, z, slope * z)


def _knn_agg(q, r, vs, wv, ts, wt, b, k, slope, sb):
    B, S, D = q.shape
    N = r.shape[1]
    C, Ci = wv.shape
    qq = jnp.sum(q * q, -1)[:, :, None]
    rr = jnp.sum(r * r, -1)[:, None, :]
    return pl.pallas_call(
        functools.partial(_knn_agg_kernel, k=k, n=N, slope=slope),
        grid=(B, S // sb),
        in_specs=[
            pl.BlockSpec((1, sb, D), lambda b, s: (b, s, 0)),
            pl.BlockSpec((1, N, D), lambda b, s: (b, 0, 0)),
            pl.BlockSpec((1, sb, 1), lambda b, s: (b, s, 0)),
            pl.BlockSpec((1, 1, N), lambda b, s: (b, 0, 0)),
            pl.BlockSpec((1, N, Ci), lambda b, s: (b, 0, 0)),
            pl.BlockSpec((C, Ci), lambda b, s: (0, 0)),
            pl.BlockSpec((1, sb, Ci), lambda b, s: (b, s, 0)),
            pl.BlockSpec((C, Ci), lambda b, s: (0, 0)),
            pl.BlockSpec((1, C), lambda b, s: (0, 0)),
        ],
        out_specs=pl.BlockSpec((1, sb, C), lambda b, s: (b, s, 0)),
        out_shape=jax.ShapeDtypeStruct((B, S, C), jnp.float32),
        compiler_params=pltpu.CompilerParams(
            dimension_semantics=("parallel", "arbitrary")),
    )(q, r, qq, rr, vs, wv, ts, wt, b.reshape(1, C))


def _fuse_kernel(g_ref, h_ref, wpa_ref, wpb_ref, bp_ref, of_ref):
    z = (_tb(g_ref[0], wpa_ref[...]) + _tb(h_ref[0], wpb_ref[...])
         + jnp.broadcast_to(bp_ref[...], (g_ref.shape[1], wpa_ref.shape[0])))
    of_ref[0] = jnp.where(z >= 0.0, z, 0.2 * z)


def _fuse(g, h, wpu, bpu):
    B, S, C = g.shape
    CO = wpu.shape[0]
    return pl.pallas_call(
        _fuse_kernel,
        grid=(B,),
        in_specs=[
            pl.BlockSpec((1, S, C), lambda b: (b, 0, 0)),
            pl.BlockSpec((1, S, C), lambda b: (b, 0, 0)),
            pl.BlockSpec((CO, C), lambda b: (0, 0)),
            pl.BlockSpec((CO, C), lambda b: (0, 0)),
            pl.BlockSpec((1, CO), lambda b: (0, 0)),
        ],
        out_specs=pl.BlockSpec((1, S, CO), lambda b: (b, 0, 0)),
        out_shape=jax.ShapeDtypeStruct((B, S, CO), jnp.float32),
        compiler_params=pltpu.CompilerParams(
            dimension_semantics=("parallel",)),
    )(g, h, wpu[:, :C], wpu[:, C:], bpu.reshape(1, CO))


def _fps_kernel(xr, yr, zr, idx_ref, nxr, nyr, nzr, *, npoint, n):
    X = xr[...]; Y = yr[...]; Z = zr[...]
    rb = X.shape[0]
    li = jax.lax.broadcasted_iota(jnp.int32, (rb, n), 1)
    si = jax.lax.broadcasted_iota(jnp.int32, (rb, npoint), 1)

    def bcast_n(v):
        return jnp.broadcast_to(v, (rb, n))

    def bcast_s(v):
        return jnp.broadcast_to(v, (rb, npoint))

    def body(i, carry):
        idxs, cxs, cys, czs, dists, far = carry
        sel = si == i
        idxs = jnp.where(sel, bcast_s(far), idxs)
        mask = li == bcast_n(far)
        cx = jnp.sum(jnp.where(mask, X, 0.0), axis=1, keepdims=True)
        cy = jnp.sum(jnp.where(mask, Y, 0.0), axis=1, keepdims=True)
        cz = jnp.sum(jnp.where(mask, Z, 0.0), axis=1, keepdims=True)
        cxs = jnp.where(sel, bcast_s(cx), cxs)
        cys = jnp.where(sel, bcast_s(cy), cys)
        czs = jnp.where(sel, bcast_s(cz), czs)
        dx = X - bcast_n(cx); dy = Y - bcast_n(cy); dz = Z - bcast_n(cz)
        d = dx * dx + dy * dy + dz * dz
        dists = jnp.minimum(dists, d)
        m = jnp.max(dists, axis=1, keepdims=True)
        far = jnp.min(jnp.where(dists == bcast_n(m), li, n),
                      axis=1, keepdims=True)
        return idxs, cxs, cys, czs, dists, far

    zs_s = X[:, :npoint] * 0.0
    init = (zs_s.astype(jnp.int32),
            zs_s, zs_s, zs_s,
            X * 0.0 + 1e10,
            (X[:, :1] * 0.0).astype(jnp.int32))
    idxs, cxs, cys, czs, _, _ = jax.lax.fori_loop(0, npoint, body, init)
    idx_ref[...] = idxs
    nxr[...] = cxs; nyr[...] = cys; nzr[...] = czs


def _fps(xs, ys, zs, npoint):
    B, N = xs.shape
    rb = B // 2
    spec_in = pl.BlockSpec((rb, N), lambda b: (b, 0))
    spec_out = pl.BlockSpec((rb, npoint), lambda b: (b, 0))
    return pl.pallas_call(
        functools.partial(_fps_kernel, npoint=npoint, n=N),
        grid=(2,),
        in_specs=[spec_in] * 3,
        out_specs=[spec_out] * 4,
        out_shape=[jax.ShapeDtypeStruct((B, npoint), jnp.int32)]
        + [jax.ShapeDtypeStruct((B, npoint), jnp.float32)] * 3,
        compiler_params=pltpu.CompilerParams(
            dimension_semantics=("parallel",)),
    )(xs, ys, zs)


def _final_kernel(f1_ref, xs_ref, w2_ref, b2_ref, wfa_ref, wfb_ref, bf_ref, o_ref):
    xs = xs_ref[0]
    f1 = f1_ref[0]
    xskip = jnp.maximum(
        jnp.dot(w2_ref[...], xs, preferred_element_type=jnp.float32)
        + b2_ref[...][:, None], 0.0)
    acc = (jnp.dot(wfa_ref[...], f1, preferred_element_type=jnp.float32)
           + jnp.dot(wfb_ref[...], xskip, preferred_element_type=jnp.float32)
           + bf_ref[...][:, None])
    o_ref[0] = jnp.where(acc >= 0.0, acc, 0.2 * acc)


def _final_stage(f1, x_strided, w2, bn2, wf, bnf):
    B = f1.shape[0]
    w2p, b2p = _fold_bn(w2, bn2)
    wfp, bfp = _fold_bn(wf, bnf)
    wfa = wfp[:, :256]
    wfb = wfp[:, 256:]
    return pl.pallas_call(
        _final_kernel,
        grid=(B,),
        in_specs=[
            pl.BlockSpec((1, 256, 256), lambda b: (b, 0, 0)),
            pl.BlockSpec((1, 64, 256), lambda b: (b, 0, 0)),
            pl.BlockSpec((1024, 64), lambda b: (0, 0)),
            pl.BlockSpec((1024,), lambda b: (0,)),
            pl.BlockSpec((512, 256), lambda b: (0, 0)),
            pl.BlockSpec((512, 1024), lambda b: (0, 0)),
            pl.BlockSpec((512,), lambda b: (0,)),
        ],
        out_specs=pl.BlockSpec((1, 512, 256), lambda b: (b, 0, 0)),
        out_shape=jax.ShapeDtypeStruct((B, 512, 256), jnp.float32),
        compiler_params=pltpu.CompilerParams(
            dimension_semantics=("parallel",)),
    )(f1, x_strided, w2p, b2p, wfa, wfb, bfp)


@jax.jit
def kernel(x, w1, bn1, w2, bn2, geo_w, geo_bn, feat_w, feat_bn, fuse_w, fuse_bn,
           gl0_w, gl0_bn, gl1_w, gl1_bn, wf, bnf):
    B, _, N = x.shape
    xyz = jnp.swapaxes(x[:, 0:3, :], 1, 2)
    g1, be1, m1, v1 = bn1
    h1 = jnp.einsum('bcn,oc->bon', x, w1)
    f_o = jax.nn.relu((h1 - m1[:, None]) * g1[:, None]
                      * jax.lax.rsqrt(v1[:, None] + _EPS) + be1[:, None])
    f_t = jnp.swapaxes(f_o, 1, 2)
    bi = jnp.arange(B)[:, None]

    wpg, bpg = _fold_bn(geo_w, geo_bn)
    wdg = wpg[:, :64]; wcdg = wpg[:, 64:] - wdg
    wpf, bpf = _fold_bn(feat_w, feat_bn)
    wdf = wpf[:, :64]; wcdf = wpf[:, 64:] - wdf
    g = _knn_agg(xyz, xyz, f_t, wdg, f_t, wcdg, bpg, k=20, slope=0.2, sb=512)
    h = _knn_agg(f_t, f_t, f_t, wdf, f_t, wcdf, bpf, k=20, slope=0.2, sb=512)
    wpu, bpu = _fold_bn(fuse_w, fuse_bn)
    out_ft = _fuse(g, h, wpu, bpu)

    xs = x[:, 0, :]; ys = x[:, 1, :]; zs = x[:, 2, :]
    fps_idx, nx, ny, nz = _fps(xs, ys, zs, 512)
    new_xyz = jnp.stack([nx, ny, nz], axis=-1)
    wp0, bp0 = _fold_bn(gl0_w, gl0_bn)
    wa0 = wp0[:, :64]
    wb0 = wp0[:, 64:] - wa0
    f0_t = _knn_agg(new_xyz, xyz, out_ft, wa0, out_ft[bi, fps_idx], wb0, bp0,
                    k=32, slope=0.0, sb=512)

    idx2, n2x, n2y, n2z = _fps(nx, ny, nz, 256)
    new_xyz2 = jnp.stack([n2x, n2y, n2z], axis=-1)
    wp1, bp1 = _fold_bn(gl1_w, gl1_bn)
    wa1 = wp1[:, :128]
    wb1 = wp1[:, 128:] - wa1
    f1_t = _knn_agg(new_xyz2, new_xyz, f0_t, wa1, f0_t[bi, idx2], wb1, bp1,
                    k=32, slope=0.0, sb=256)

    f1 = jnp.swapaxes(f1_t, 1, 2)
    stride = N // 256
    return _final_stage(f1, f_o[:, :, ::stride], w2, bn2, wf, bnf)


# speedup vs baseline: 8.1543x; 1.0349x over previous
import functools
import jax
import jax.numpy as jnp
import numpy as np
from jax.experimental import pallas as pl
from jax.experimental.pallas import tpu as pltpu

_EPS = 1e-5


def _fold_bn(w, bnp):
    g, b, m, v = bnp
    s = g * jax.lax.rsqrt(v + _EPS)
    return w * s[:, None], b - m * s


def _tb(a, b):
    return jax.lax.dot_general(a, b, (((1,), (1,)), ((), ())),
                               preferred_element_type=jnp.float32)


def _knn_agg_kernel(q_ref, r_ref, qq_ref, rr_ref, vs_ref, wv_ref, ts_ref,
                    wt_ref, b_ref, o_ref, *, k, n, slope):
    q = q_ref[0]
    r = r_ref[0]
    v = _tb(vs_ref[0], wv_ref[...])
    tq = _tb(ts_ref[0], wt_ref[...]) + jnp.broadcast_to(
        b_ref[...], (ts_ref.shape[1], wt_ref.shape[0]))
    sb = q.shape[0]
    c = v.shape[1]
    qq = qq_ref[0]
    rr = rr_ref[0]
    qr = jax.lax.dot_general(q, r, (((1,), (1,)), ((), ())),
                             preferred_element_type=jnp.float32)
    d2 = (jnp.broadcast_to(qq, (sb, n)) - 2.0 * qr) + jnp.broadcast_to(rr, (sb, n))
    li = jax.lax.broadcasted_iota(jnp.int32, (sb, n), 1)

    def body(_, carry):
        d2, out = carry
        m = jnp.min(d2, axis=1, keepdims=True)
        eqm = d2 == jnp.broadcast_to(m, (sb, n))
        mi = jnp.min(jnp.where(eqm, li, n), axis=1, keepdims=True)
        oh = li == jnp.broadcast_to(mi, (sb, n))
        d2 = jnp.where(oh, 3.4e38, d2)
        contrib = jax.lax.dot_general(
            oh.astype(jnp.float32), v, (((1,), (0,)), ((), ())),
            preferred_element_type=jnp.float32)
        return d2, jnp.maximum(out, contrib)

    out0 = qr[:, :c] * 0.0 - 3e38
    _, mx = jax.lax.fori_loop(0, k, body, (d2, out0))
    z = mx + tq
    o_ref[0] = jnp.where(z >= 0.0, z, slope * z)


def _knn_agg(q, r, vs, wv, ts, wt, b, k, slope, sb):
    B, S, D = q.shape
    N = r.shape[1]
    C, Ci = wv.shape
    qq = jnp.sum(q * q, -1)[:, :, None]
    rr = jnp.sum(r * r, -1)[:, None, :]
    return pl.pallas_call(
        functools.partial(_knn_agg_kernel, k=k, n=N, slope=slope),
        grid=(B, S // sb),
        in_specs=[
            pl.BlockSpec((1, sb, D), lambda b, s: (b, s, 0)),
            pl.BlockSpec((1, N, D), lambda b, s: (b, 0, 0)),
            pl.BlockSpec((1, sb, 1), lambda b, s: (b, s, 0)),
            pl.BlockSpec((1, 1, N), lambda b, s: (b, 0, 0)),
            pl.BlockSpec((1, N, Ci), lambda b, s: (b, 0, 0)),
            pl.BlockSpec((C, Ci), lambda b, s: (0, 0)),
            pl.BlockSpec((1, sb, Ci), lambda b, s: (b, s, 0)),
            pl.BlockSpec((C, Ci), lambda b, s: (0, 0)),
            pl.BlockSpec((1, C), lambda b, s: (0, 0)),
        ],
        out_specs=pl.BlockSpec((1, sb, C), lambda b, s: (b, s, 0)),
        out_shape=jax.ShapeDtypeStruct((B, S, C), jnp.float32),
        compiler_params=pltpu.CompilerParams(
            dimension_semantics=("parallel", "arbitrary")),
    )(q, r, qq, rr, vs, wv, ts, wt, b.reshape(1, C))


def _fuse_kernel(g_ref, h_ref, wpa_ref, wpb_ref, bp_ref, of_ref):
    z = (_tb(g_ref[0], wpa_ref[...]) + _tb(h_ref[0], wpb_ref[...])
         + jnp.broadcast_to(bp_ref[...], (g_ref.shape[1], wpa_ref.shape[0])))
    of_ref[0] = jnp.where(z >= 0.0, z, 0.2 * z)


def _fuse(g, h, wpu, bpu):
    B, S, C = g.shape
    CO = wpu.shape[0]
    return pl.pallas_call(
        _fuse_kernel,
        grid=(B,),
        in_specs=[
            pl.BlockSpec((1, S, C), lambda b: (b, 0, 0)),
            pl.BlockSpec((1, S, C), lambda b: (b, 0, 0)),
            pl.BlockSpec((CO, C), lambda b: (0, 0)),
            pl.BlockSpec((CO, C), lambda b: (0, 0)),
            pl.BlockSpec((1, CO), lambda b: (0, 0)),
        ],
        out_specs=pl.BlockSpec((1, S, CO), lambda b: (b, 0, 0)),
        out_shape=jax.ShapeDtypeStruct((B, S, CO), jnp.float32),
        compiler_params=pltpu.CompilerParams(
            dimension_semantics=("parallel",)),
    )(g, h, wpu[:, :C], wpu[:, C:], bpu.reshape(1, CO))


def _fps_kernel(xr, yr, zr, idx_ref, nxr, nyr, nzr, *, npoint, n):
    X = xr[...]; Y = yr[...]; Z = zr[...]
    rb = X.shape[0]
    li = jax.lax.broadcasted_iota(jnp.int32, (rb, n), 1)
    si = jax.lax.broadcasted_iota(jnp.int32, (rb, npoint), 1)

    def bcast_n(v):
        return jnp.broadcast_to(v, (rb, n))

    def bcast_s(v):
        return jnp.broadcast_to(v, (rb, npoint))

    def body(i, carry):
        idxs, cxs, cys, czs, dists, far = carry
        sel = si == i
        idxs = jnp.where(sel, bcast_s(far), idxs)
        mask = li == bcast_n(far)
        cx = jnp.sum(jnp.where(mask, X, 0.0), axis=1, keepdims=True)
        cy = jnp.sum(jnp.where(mask, Y, 0.0), axis=1, keepdims=True)
        cz = jnp.sum(jnp.where(mask, Z, 0.0), axis=1, keepdims=True)
        cxs = jnp.where(sel, bcast_s(cx), cxs)
        cys = jnp.where(sel, bcast_s(cy), cys)
        czs = jnp.where(sel, bcast_s(cz), czs)
        dx = X - bcast_n(cx); dy = Y - bcast_n(cy); dz = Z - bcast_n(cz)
        d = dx * dx + dy * dy + dz * dz
        dists = jnp.minimum(dists, d)
        m = jnp.max(dists, axis=1, keepdims=True)
        far = jnp.min(jnp.where(dists == bcast_n(m), li, n),
                      axis=1, keepdims=True)
        return idxs, cxs, cys, czs, dists, far

    zs_s = X[:, :npoint] * 0.0
    init = (zs_s.astype(jnp.int32),
            zs_s, zs_s, zs_s,
            X * 0.0 + 1e10,
            (X[:, :1] * 0.0).astype(jnp.int32))
    idxs, cxs, cys, czs, _, _ = jax.lax.fori_loop(0, npoint, body, init)
    idx_ref[...] = idxs
    nxr[...] = cxs; nyr[...] = cys; nzr[...] = czs


def _fps(xs, ys, zs, npoint):
    B, N = xs.shape
    rb = B // 2
    spec_in = pl.BlockSpec((rb, N), lambda b: (b, 0))
    spec_out = pl.BlockSpec((rb, npoint), lambda b: (b, 0))
    return pl.pallas_call(
        functools.partial(_fps_kernel, npoint=npoint, n=N),
        grid=(2,),
        in_specs=[spec_in] * 3,
        out_specs=[spec_out] * 4,
        out_shape=[jax.ShapeDtypeStruct((B, npoint), jnp.int32)]
        + [jax.ShapeDtypeStruct((B, npoint), jnp.float32)] * 3,
        compiler_params=pltpu.CompilerParams(
            dimension_semantics=("parallel",)),
    )(xs, ys, zs)


def _final_kernel(f1_ref, xs_ref, w2_ref, b2_ref, wfa_ref, wfb_ref, bf_ref, o_ref):
    xs = xs_ref[0]
    f1 = f1_ref[0]
    xskip = jnp.maximum(
        jnp.dot(w2_ref[...], xs, preferred_element_type=jnp.float32)
        + b2_ref[...][:, None], 0.0)
    acc = (jnp.dot(wfa_ref[...], f1, preferred_element_type=jnp.float32)
           + jnp.dot(wfb_ref[...], xskip, preferred_element_type=jnp.float32)
           + bf_ref[...][:, None])
    o_ref[0] = jnp.where(acc >= 0.0, acc, 0.2 * acc)


def _final_stage(f1, x_strided, w2, bn2, wf, bnf):
    B = f1.shape[0]
    w2p, b2p = _fold_bn(w2, bn2)
    wfp, bfp = _fold_bn(wf, bnf)
    wfa = wfp[:, :256]
    wfb = wfp[:, 256:]
    return pl.pallas_call(
        _final_kernel,
        grid=(B,),
        in_specs=[
            pl.BlockSpec((1, 256, 256), lambda b: (b, 0, 0)),
            pl.BlockSpec((1, 64, 256), lambda b: (b, 0, 0)),
            pl.BlockSpec((1024, 64), lambda b: (0, 0)),
            pl.BlockSpec((1024,), lambda b: (0,)),
            pl.BlockSpec((512, 256), lambda b: (0, 0)),
            pl.BlockSpec((512, 1024), lambda b: (0, 0)),
            pl.BlockSpec((512,), lambda b: (0,)),
        ],
        out_specs=pl.BlockSpec((1, 512, 256), lambda b: (b, 0, 0)),
        out_shape=jax.ShapeDtypeStruct((B, 512, 256), jnp.float32),
        compiler_params=pltpu.CompilerParams(
            dimension_semantics=("parallel",)),
    )(f1, x_strided, w2p, b2p, wfa, wfb, bfp)


@jax.jit
def kernel(x, w1, bn1, w2, bn2, geo_w, geo_bn, feat_w, feat_bn, fuse_w, fuse_bn,
           gl0_w, gl0_bn, gl1_w, gl1_bn, wf, bnf):
    B, _, N = x.shape
    xyz = jnp.swapaxes(x[:, 0:3, :], 1, 2)
    g1, be1, m1, v1 = bn1
    h1 = jnp.einsum('bcn,oc->bon', x, w1)
    f_o = jax.nn.relu((h1 - m1[:, None]) * g1[:, None]
                      * jax.lax.rsqrt(v1[:, None] + _EPS) + be1[:, None])
    f_t = jnp.swapaxes(f_o, 1, 2)
    bi = jnp.arange(B)[:, None]

    wpg, bpg = _fold_bn(geo_w, geo_bn)
    wdg = wpg[:, :64]; wcdg = wpg[:, 64:] - wdg
    wpf, bpf = _fold_bn(feat_w, feat_bn)
    wdf = wpf[:, :64]; wcdf = wpf[:, 64:] - wdf
    g = _knn_agg(xyz, xyz, f_t, wdg, f_t, wcdg, bpg, k=20, slope=0.2, sb=1024)
    h = _knn_agg(f_t, f_t, f_t, wdf, f_t, wcdf, bpf, k=20, slope=0.2, sb=1024)
    wpu, bpu = _fold_bn(fuse_w, fuse_bn)
    out_ft = _fuse(g, h, wpu, bpu)

    xs = x[:, 0, :]; ys = x[:, 1, :]; zs = x[:, 2, :]
    fps_idx, nx, ny, nz = _fps(xs, ys, zs, 512)
    new_xyz = jnp.stack([nx, ny, nz], axis=-1)
    wp0, bp0 = _fold_bn(gl0_w, gl0_bn)
    wa0 = wp0[:, :64]
    wb0 = wp0[:, 64:] - wa0
    f0_t = _knn_agg(new_xyz, xyz, out_ft, wa0, out_ft[bi, fps_idx], wb0, bp0,
                    k=32, slope=0.0, sb=512)

    idx2, n2x, n2y, n2z = _fps(nx, ny, nz, 256)
    new_xyz2 = jnp.stack([n2x, n2y, n2z], axis=-1)
    wp1, bp1 = _fold_bn(gl1_w, gl1_bn)
    wa1 = wp1[:, :128]
    wb1 = wp1[:, 128:] - wa1
    f1_t = _knn_agg(new_xyz2, new_xyz, f0_t, wa1, f0_t[bi, idx2], wb1, bp1,
                    k=32, slope=0.0, sb=256)

    f1 = jnp.swapaxes(f1_t, 1, 2)
    stride = N // 256
    return _final_stage(f1, f_o[:, :, ::stride], w2, bn2, wf, bnf)


# speedup vs baseline: 8.3196x; 1.0203x over previous
import functools
import jax
import jax.numpy as jnp
import numpy as np
from jax.experimental import pallas as pl
from jax.experimental.pallas import tpu as pltpu

_EPS = 1e-5


def _fold_bn(w, bnp):
    g, b, m, v = bnp
    s = g * jax.lax.rsqrt(v + _EPS)
    return w * s[:, None], b - m * s


def _tb(a, b):
    return jax.lax.dot_general(a, b, (((1,), (1,)), ((), ())),
                               preferred_element_type=jnp.float32)


def _knn_agg_kernel(q_ref, r_ref, qq_ref, rr_ref, vs_ref, wv_ref, ts_ref,
                    wt_ref, b_ref, o_ref, *, k, n, slope):
    q = q_ref[0]
    r = r_ref[0]
    v = _tb(vs_ref[0], wv_ref[...])
    tq = _tb(ts_ref[0], wt_ref[...]) + jnp.broadcast_to(
        b_ref[...], (ts_ref.shape[1], wt_ref.shape[0]))
    sb = q.shape[0]
    c = v.shape[1]
    qq = qq_ref[0]
    rr = rr_ref[0]
    qr = jax.lax.dot_general(q, r, (((1,), (1,)), ((), ())),
                             preferred_element_type=jnp.float32)
    d2 = (jnp.broadcast_to(qq, (sb, n)) - 2.0 * qr) + jnp.broadcast_to(rr, (sb, n))
    li = jax.lax.broadcasted_iota(jnp.int32, (sb, n), 1)

    def body(_, carry):
        d2, out = carry
        mi = jnp.argmin(d2, axis=1, keepdims=True).astype(jnp.int32)
        oh = li == jnp.broadcast_to(mi, (sb, n))
        d2 = jnp.where(oh, 3.4e38, d2)
        contrib = jax.lax.dot_general(
            oh.astype(jnp.float32), v, (((1,), (0,)), ((), ())),
            preferred_element_type=jnp.float32)
        return d2, jnp.maximum(out, contrib)

    out0 = qr[:, :c] * 0.0 - 3e38
    _, mx = jax.lax.fori_loop(0, k, body, (d2, out0))
    z = mx + tq
    o_ref[0] = jnp.where(z >= 0.0, z, slope * z)


def _knn_agg(q, r, vs, wv, ts, wt, b, k, slope, sb):
    B, S, D = q.shape
    N = r.shape[1]
    C, Ci = wv.shape
    qq = jnp.sum(q * q, -1)[:, :, None]
    rr = jnp.sum(r * r, -1)[:, None, :]
    return pl.pallas_call(
        functools.partial(_knn_agg_kernel, k=k, n=N, slope=slope),
        grid=(B, S // sb),
        in_specs=[
            pl.BlockSpec((1, sb, D), lambda b, s: (b, s, 0)),
            pl.BlockSpec((1, N, D), lambda b, s: (b, 0, 0)),
            pl.BlockSpec((1, sb, 1), lambda b, s: (b, s, 0)),
            pl.BlockSpec((1, 1, N), lambda b, s: (b, 0, 0)),
            pl.BlockSpec((1, N, Ci), lambda b, s: (b, 0, 0)),
            pl.BlockSpec((C, Ci), lambda b, s: (0, 0)),
            pl.BlockSpec((1, sb, Ci), lambda b, s: (b, s, 0)),
            pl.BlockSpec((C, Ci), lambda b, s: (0, 0)),
            pl.BlockSpec((1, C), lambda b, s: (0, 0)),
        ],
        out_specs=pl.BlockSpec((1, sb, C), lambda b, s: (b, s, 0)),
        out_shape=jax.ShapeDtypeStruct((B, S, C), jnp.float32),
        compiler_params=pltpu.CompilerParams(
            dimension_semantics=("parallel", "arbitrary")),
    )(q, r, qq, rr, vs, wv, ts, wt, b.reshape(1, C))


def _fuse_kernel(g_ref, h_ref, wpa_ref, wpb_ref, bp_ref, of_ref):
    z = (_tb(g_ref[0], wpa_ref[...]) + _tb(h_ref[0], wpb_ref[...])
         + jnp.broadcast_to(bp_ref[...], (g_ref.shape[1], wpa_ref.shape[0])))
    of_ref[0] = jnp.where(z >= 0.0, z, 0.2 * z)


def _fuse(g, h, wpu, bpu):
    B, S, C = g.shape
    CO = wpu.shape[0]
    return pl.pallas_call(
        _fuse_kernel,
        grid=(B,),
        in_specs=[
            pl.BlockSpec((1, S, C), lambda b: (b, 0, 0)),
            pl.BlockSpec((1, S, C), lambda b: (b, 0, 0)),
            pl.BlockSpec((CO, C), lambda b: (0, 0)),
            pl.BlockSpec((CO, C), lambda b: (0, 0)),
            pl.BlockSpec((1, CO), lambda b: (0, 0)),
        ],
        out_specs=pl.BlockSpec((1, S, CO), lambda b: (b, 0, 0)),
        out_shape=jax.ShapeDtypeStruct((B, S, CO), jnp.float32),
        compiler_params=pltpu.CompilerParams(
            dimension_semantics=("parallel",)),
    )(g, h, wpu[:, :C], wpu[:, C:], bpu.reshape(1, CO))


def _fps_kernel(xr, yr, zr, idx_ref, nxr, nyr, nzr, *, npoint, n):
    X = xr[...]; Y = yr[...]; Z = zr[...]
    rb = X.shape[0]
    li = jax.lax.broadcasted_iota(jnp.int32, (rb, n), 1)
    si = jax.lax.broadcasted_iota(jnp.int32, (rb, npoint), 1)

    def bcast_n(v):
        return jnp.broadcast_to(v, (rb, n))

    def bcast_s(v):
        return jnp.broadcast_to(v, (rb, npoint))

    def body(i, carry):
        idxs, cxs, cys, czs, dists, far = carry
        sel = si == i
        idxs = jnp.where(sel, bcast_s(far), idxs)
        mask = li == bcast_n(far)
        cx = jnp.sum(jnp.where(mask, X, 0.0), axis=1, keepdims=True)
        cy = jnp.sum(jnp.where(mask, Y, 0.0), axis=1, keepdims=True)
        cz = jnp.sum(jnp.where(mask, Z, 0.0), axis=1, keepdims=True)
        cxs = jnp.where(sel, bcast_s(cx), cxs)
        cys = jnp.where(sel, bcast_s(cy), cys)
        czs = jnp.where(sel, bcast_s(cz), czs)
        dx = X - bcast_n(cx); dy = Y - bcast_n(cy); dz = Z - bcast_n(cz)
        d = dx * dx + dy * dy + dz * dz
        dists = jnp.minimum(dists, d)
        m = jnp.max(dists, axis=1, keepdims=True)
        far = jnp.min(jnp.where(dists == bcast_n(m), li, n),
                      axis=1, keepdims=True)
        return idxs, cxs, cys, czs, dists, far

    zs_s = X[:, :npoint] * 0.0
    init = (zs_s.astype(jnp.int32),
            zs_s, zs_s, zs_s,
            X * 0.0 + 1e10,
            (X[:, :1] * 0.0).astype(jnp.int32))
    idxs, cxs, cys, czs, _, _ = jax.lax.fori_loop(0, npoint, body, init)
    idx_ref[...] = idxs
    nxr[...] = cxs; nyr[...] = cys; nzr[...] = czs


def _fps(xs, ys, zs, npoint):
    B, N = xs.shape
    rb = B // 2
    spec_in = pl.BlockSpec((rb, N), lambda b: (b, 0))
    spec_out = pl.BlockSpec((rb, npoint), lambda b: (b, 0))
    return pl.pallas_call(
        functools.partial(_fps_kernel, npoint=npoint, n=N),
        grid=(2,),
        in_specs=[spec_in] * 3,
        out_specs=[spec_out] * 4,
        out_shape=[jax.ShapeDtypeStruct((B, npoint), jnp.int32)]
        + [jax.ShapeDtypeStruct((B, npoint), jnp.float32)] * 3,
        compiler_params=pltpu.CompilerParams(
            dimension_semantics=("parallel",)),
    )(xs, ys, zs)


def _final_kernel(f1_ref, xs_ref, w2_ref, b2_ref, wfa_ref, wfb_ref, bf_ref, o_ref):
    xs = xs_ref[0]
    f1 = f1_ref[0]
    xskip = jnp.maximum(
        jnp.dot(w2_ref[...], xs, preferred_element_type=jnp.float32)
        + b2_ref[...][:, None], 0.0)
    acc = (jnp.dot(wfa_ref[...], f1, preferred_element_type=jnp.float32)
           + jnp.dot(wfb_ref[...], xskip, preferred_element_type=jnp.float32)
           + bf_ref[...][:, None])
    o_ref[0] = jnp.where(acc >= 0.0, acc, 0.2 * acc)


def _final_stage(f1, x_strided, w2, bn2, wf, bnf):
    B = f1.shape[0]
    w2p, b2p = _fold_bn(w2, bn2)
    wfp, bfp = _fold_bn(wf, bnf)
    wfa = wfp[:, :256]
    wfb = wfp[:, 256:]
    return pl.pallas_call(
        _final_kernel,
        grid=(B,),
        in_specs=[
            pl.BlockSpec((1, 256, 256), lambda b: (b, 0, 0)),
            pl.BlockSpec((1, 64, 256), lambda b: (b, 0, 0)),
            pl.BlockSpec((1024, 64), lambda b: (0, 0)),
            pl.BlockSpec((1024,), lambda b: (0,)),
            pl.BlockSpec((512, 256), lambda b: (0, 0)),
            pl.BlockSpec((512, 1024), lambda b: (0, 0)),
            pl.BlockSpec((512,), lambda b: (0,)),
        ],
        out_specs=pl.BlockSpec((1, 512, 256), lambda b: (b, 0, 0)),
        out_shape=jax.ShapeDtypeStruct((B, 512, 256), jnp.float32),
        compiler_params=pltpu.CompilerParams(
            dimension_semantics=("parallel",)),
    )(f1, x_strided, w2p, b2p, wfa, wfb, bfp)


@jax.jit
def kernel(x, w1, bn1, w2, bn2, geo_w, geo_bn, feat_w, feat_bn, fuse_w, fuse_bn,
           gl0_w, gl0_bn, gl1_w, gl1_bn, wf, bnf):
    B, _, N = x.shape
    xyz = jnp.swapaxes(x[:, 0:3, :], 1, 2)
    g1, be1, m1, v1 = bn1
    h1 = jnp.einsum('bcn,oc->bon', x, w1)
    f_o = jax.nn.relu((h1 - m1[:, None]) * g1[:, None]
                      * jax.lax.rsqrt(v1[:, None] + _EPS) + be1[:, None])
    f_t = jnp.swapaxes(f_o, 1, 2)
    bi = jnp.arange(B)[:, None]

    wpg, bpg = _fold_bn(geo_w, geo_bn)
    wdg = wpg[:, :64]; wcdg = wpg[:, 64:] - wdg
    wpf, bpf = _fold_bn(feat_w, feat_bn)
    wdf = wpf[:, :64]; wcdf = wpf[:, 64:] - wdf
    g = _knn_agg(xyz, xyz, f_t, wdg, f_t, wcdg, bpg, k=20, slope=0.2, sb=1024)
    h = _knn_agg(f_t, f_t, f_t, wdf, f_t, wcdf, bpf, k=20, slope=0.2, sb=1024)
    wpu, bpu = _fold_bn(fuse_w, fuse_bn)
    out_ft = _fuse(g, h, wpu, bpu)

    xs = x[:, 0, :]; ys = x[:, 1, :]; zs = x[:, 2, :]
    fps_idx, nx, ny, nz = _fps(xs, ys, zs, 512)
    new_xyz = jnp.stack([nx, ny, nz], axis=-1)
    wp0, bp0 = _fold_bn(gl0_w, gl0_bn)
    wa0 = wp0[:, :64]
    wb0 = wp0[:, 64:] - wa0
    f0_t = _knn_agg(new_xyz, xyz, out_ft, wa0, out_ft[bi, fps_idx], wb0, bp0,
                    k=32, slope=0.0, sb=512)

    idx2, n2x, n2y, n2z = _fps(nx, ny, nz, 256)
    new_xyz2 = jnp.stack([n2x, n2y, n2z], axis=-1)
    wp1, bp1 = _fold_bn(gl1_w, gl1_bn)
    wa1 = wp1[:, :128]
    wb1 = wp1[:, 128:] - wa1
    f1_t = _knn_agg(new_xyz2, new_xyz, f0_t, wa1, f0_t[bi, idx2], wb1, bp1,
                    k=32, slope=0.0, sb=256)

    f1 = jnp.swapaxes(f1_t, 1, 2)
    stride = N // 256
    return _final_stage(f1, f_o[:, :, ::stride], w2, bn2, wf, bnf)


# speedup vs baseline: 8.3211x; 1.0002x over previous
import functools
import jax
import jax.numpy as jnp
from jax.experimental import pallas as pl
from jax.experimental.pallas import tpu as pltpu

_EPS = 1e-5


def _fold_bn(w, bnp):
    g, b, m, v = bnp
    s = g * jax.lax.rsqrt(v + _EPS)
    return w * s[:, None], b - m * s


def _tb(a, b):
    return jax.lax.dot_general(a, b, (((1,), (1,)), ((), ())),
                               preferred_element_type=jnp.float32)


def _knn_agg_kernel(q_ref, r_ref, qq_ref, rr_ref, vs_ref, wv_ref, ts_ref,
                    wt_ref, b_ref, o_ref, *, k, n, slope):
    q = q_ref[0]
    r = r_ref[0]
    v = _tb(vs_ref[0], wv_ref[...])
    tq = _tb(ts_ref[0], wt_ref[...]) + jnp.broadcast_to(
        b_ref[...], (ts_ref.shape[1], wt_ref.shape[0]))
    sb = q.shape[0]
    c = v.shape[1]
    qq = qq_ref[0]
    rr = rr_ref[0]
    qr = jax.lax.dot_general(q, r, (((1,), (1,)), ((), ())),
                             preferred_element_type=jnp.float32)
    d2 = (jnp.broadcast_to(qq, (sb, n)) - 2.0 * qr) + jnp.broadcast_to(rr, (sb, n))
    li = jax.lax.broadcasted_iota(jnp.int32, (sb, n), 1)

    def body(_, carry):
        d2, out = carry
        mi = jnp.argmin(d2, axis=1, keepdims=True).astype(jnp.int32)
        oh = li == jnp.broadcast_to(mi, (sb, n))
        d2 = jnp.where(oh, 3.4e38, d2)
        contrib = jax.lax.dot_general(
            oh.astype(jnp.float32), v, (((1,), (0,)), ((), ())),
            preferred_element_type=jnp.float32)
        return d2, jnp.maximum(out, contrib)

    out0 = qr[:, :c] * 0.0 - 3e38
    _, mx = jax.lax.fori_loop(0, k, body, (d2, out0))
    z = mx + tq
    o_ref[0] = jnp.where(z >= 0.0, z, slope * z)


def _knn_agg(q, r, vs, wv, ts, wt, b, k, slope, sb):
    B, S, D = q.shape
    N = r.shape[1]
    C, Ci = wv.shape
    qq = jnp.sum(q * q, -1)[:, :, None]
    rr = jnp.sum(r * r, -1)[:, None, :]
    return pl.pallas_call(
        functools.partial(_knn_agg_kernel, k=k, n=N, slope=slope),
        grid=(B, S // sb),
        in_specs=[
            pl.BlockSpec((1, sb, D), lambda b, s: (b, s, 0)),
            pl.BlockSpec((1, N, D), lambda b, s: (b, 0, 0)),
            pl.BlockSpec((1, sb, 1), lambda b, s: (b, s, 0)),
            pl.BlockSpec((1, 1, N), lambda b, s: (b, 0, 0)),
            pl.BlockSpec((1, N, Ci), lambda b, s: (b, 0, 0)),
            pl.BlockSpec((C, Ci), lambda b, s: (0, 0)),
            pl.BlockSpec((1, sb, Ci), lambda b, s: (b, s, 0)),
            pl.BlockSpec((C, Ci), lambda b, s: (0, 0)),
            pl.BlockSpec((1, C), lambda b, s: (0, 0)),
        ],
        out_specs=pl.BlockSpec((1, sb, C), lambda b, s: (b, s, 0)),
        out_shape=jax.ShapeDtypeStruct((B, S, C), jnp.float32),
        compiler_params=pltpu.CompilerParams(
            dimension_semantics=("parallel", "arbitrary")),
    )(q, r, qq, rr, vs, wv, ts, wt, b.reshape(1, C))


def _fuse_kernel(g_ref, h_ref, wpa_ref, wpb_ref, bp_ref, of_ref):
    z = (_tb(g_ref[0], wpa_ref[...]) + _tb(h_ref[0], wpb_ref[...])
         + jnp.broadcast_to(bp_ref[...], (g_ref.shape[1], wpa_ref.shape[0])))
    of_ref[0] = jnp.where(z >= 0.0, z, 0.2 * z)


def _fuse(g, h, wpu, bpu):
    B, S, C = g.shape
    CO = wpu.shape[0]
    return pl.pallas_call(
        _fuse_kernel,
        grid=(B,),
        in_specs=[
            pl.BlockSpec((1, S, C), lambda b: (b, 0, 0)),
            pl.BlockSpec((1, S, C), lambda b: (b, 0, 0)),
            pl.BlockSpec((CO, C), lambda b: (0, 0)),
            pl.BlockSpec((CO, C), lambda b: (0, 0)),
            pl.BlockSpec((1, CO), lambda b: (0, 0)),
        ],
        out_specs=pl.BlockSpec((1, S, CO), lambda b: (b, 0, 0)),
        out_shape=jax.ShapeDtypeStruct((B, S, CO), jnp.float32),
        compiler_params=pltpu.CompilerParams(
            dimension_semantics=("parallel",)),
    )(g, h, wpu[:, :C], wpu[:, C:], bpu.reshape(1, CO))


def _fps_kernel(xr, yr, zr, idx_ref, nxr, nyr, nzr, *, npoint, n):
    X = xr[...]; Y = yr[...]; Z = zr[...]
    rb = X.shape[0]
    li = jax.lax.broadcasted_iota(jnp.int32, (rb, n), 1)
    si = jax.lax.broadcasted_iota(jnp.int32, (rb, npoint), 1)

    def bcast_n(v):
        return jnp.broadcast_to(v, (rb, n))

    def bcast_s(v):
        return jnp.broadcast_to(v, (rb, npoint))

    def body(i, carry):
        idxs, cxs, cys, czs, dists, far = carry
        sel = si == i
        idxs = jnp.where(sel, bcast_s(far), idxs)
        mask = li == bcast_n(far)
        cx = jnp.sum(jnp.where(mask, X, 0.0), axis=1, keepdims=True)
        cy = jnp.sum(jnp.where(mask, Y, 0.0), axis=1, keepdims=True)
        cz = jnp.sum(jnp.where(mask, Z, 0.0), axis=1, keepdims=True)
        cxs = jnp.where(sel, bcast_s(cx), cxs)
        cys = jnp.where(sel, bcast_s(cy), cys)
        czs = jnp.where(sel, bcast_s(cz), czs)
        dx = X - bcast_n(cx); dy = Y - bcast_n(cy); dz = Z - bcast_n(cz)
        d = dx * dx + dy * dy + dz * dz
        dists = jnp.minimum(dists, d)
        m = jnp.max(dists, axis=1, keepdims=True)
        far = jnp.min(jnp.where(dists == bcast_n(m), li, n),
                      axis=1, keepdims=True)
        return idxs, cxs, cys, czs, dists, far

    zs_s = X[:, :npoint] * 0.0
    init = (zs_s.astype(jnp.int32),
            zs_s, zs_s, zs_s,
            X * 0.0 + 1e10,
            (X[:, :1] * 0.0).astype(jnp.int32))
    idxs, cxs, cys, czs, _, _ = jax.lax.fori_loop(0, npoint, body, init)
    idx_ref[...] = idxs
    nxr[...] = cxs; nyr[...] = cys; nzr[...] = czs


def _fps(xs, ys, zs, npoint):
    B, N = xs.shape
    rb = B // 2
    spec_in = pl.BlockSpec((rb, N), lambda b: (b, 0))
    spec_out = pl.BlockSpec((rb, npoint), lambda b: (b, 0))
    return pl.pallas_call(
        functools.partial(_fps_kernel, npoint=npoint, n=N),
        grid=(2,),
        in_specs=[spec_in] * 3,
        out_specs=[spec_out] * 4,
        out_shape=[jax.ShapeDtypeStruct((B, npoint), jnp.int32)]
        + [jax.ShapeDtypeStruct((B, npoint), jnp.float32)] * 3,
        compiler_params=pltpu.CompilerParams(
            dimension_semantics=("parallel",)),
    )(xs, ys, zs)


def _final_kernel(f1_ref, xs_ref, w2_ref, b2_ref, wfa_ref, wfb_ref, bf_ref, o_ref):
    xs = xs_ref[0]
    f1 = f1_ref[0]
    xskip = jnp.maximum(
        jnp.dot(w2_ref[...], xs, preferred_element_type=jnp.float32)
        + b2_ref[...][:, None], 0.0)
    acc = (jnp.dot(wfa_ref[...], f1, preferred_element_type=jnp.float32)
           + jnp.dot(wfb_ref[...], xskip, preferred_element_type=jnp.float32)
           + bf_ref[...][:, None])
    o_ref[0] = jnp.where(acc >= 0.0, acc, 0.2 * acc)


def _final_stage(f1, x_strided, w2, bn2, wf, bnf):
    B = f1.shape[0]
    w2p, b2p = _fold_bn(w2, bn2)
    wfp, bfp = _fold_bn(wf, bnf)
    wfa = wfp[:, :256]
    wfb = wfp[:, 256:]
    return pl.pallas_call(
        _final_kernel,
        grid=(B,),
        in_specs=[
            pl.BlockSpec((1, 256, 256), lambda b: (b, 0, 0)),
            pl.BlockSpec((1, 64, 256), lambda b: (b, 0, 0)),
            pl.BlockSpec((1024, 64), lambda b: (0, 0)),
            pl.BlockSpec((1024,), lambda b: (0,)),
            pl.BlockSpec((512, 256), lambda b: (0, 0)),
            pl.BlockSpec((512, 1024), lambda b: (0, 0)),
            pl.BlockSpec((512,), lambda b: (0,)),
        ],
        out_specs=pl.BlockSpec((1, 512, 256), lambda b: (b, 0, 0)),
        out_shape=jax.ShapeDtypeStruct((B, 512, 256), jnp.float32),
        compiler_params=pltpu.CompilerParams(
            dimension_semantics=("parallel",)),
    )(f1, x_strided, w2p, b2p, wfa, wfb, bfp)


@jax.jit
def kernel(x, w1, bn1, w2, bn2, geo_w, geo_bn, feat_w, feat_bn, fuse_w, fuse_bn,
           gl0_w, gl0_bn, gl1_w, gl1_bn, wf, bnf):
    B, _, N = x.shape
    xyz = jnp.swapaxes(x[:, 0:3, :], 1, 2)
    g1, be1, m1, v1 = bn1
    h1 = jnp.einsum('bcn,oc->bon', x, w1)
    f_o = jax.nn.relu((h1 - m1[:, None]) * g1[:, None]
                      * jax.lax.rsqrt(v1[:, None] + _EPS) + be1[:, None])
    f_t = jnp.swapaxes(f_o, 1, 2)
    bi = jnp.arange(B)[:, None]

    wpg, bpg = _fold_bn(geo_w, geo_bn)
    wdg = wpg[:, :64]; wcdg = wpg[:, 64:] - wdg
    wpf, bpf = _fold_bn(feat_w, feat_bn)
    wdf = wpf[:, :64]; wcdf = wpf[:, 64:] - wdf
    g = _knn_agg(xyz, xyz, f_t, wdg, f_t, wcdg, bpg, k=20, slope=0.2, sb=1024)
    h = _knn_agg(f_t, f_t, f_t, wdf, f_t, wcdf, bpf, k=20, slope=0.2, sb=1024)
    wpu, bpu = _fold_bn(fuse_w, fuse_bn)
    out_ft = _fuse(g, h, wpu, bpu)

    xs = x[:, 0, :]; ys = x[:, 1, :]; zs = x[:, 2, :]
    fps_idx, nx, ny, nz = _fps(xs, ys, zs, 512)
    new_xyz = jnp.stack([nx, ny, nz], axis=-1)
    wp0, bp0 = _fold_bn(gl0_w, gl0_bn)
    wa0 = wp0[:, :64]
    wb0 = wp0[:, 64:] - wa0
    f0_t = _knn_agg(new_xyz, xyz, out_ft, wa0, out_ft[bi, fps_idx], wb0, bp0,
                    k=32, slope=0.0, sb=512)

    idx2, n2x, n2y, n2z = _fps(nx, ny, nz, 256)
    new_xyz2 = jnp.stack([n2x, n2y, n2z], axis=-1)
    wp1, bp1 = _fold_bn(gl1_w, gl1_bn)
    wa1 = wp1[:, :128]
    wb1 = wp1[:, 128:] - wa1
    f1_t = _knn_agg(new_xyz2, new_xyz, f0_t, wa1, f0_t[bi, idx2], wb1, bp1,
                    k=32, slope=0.0, sb=256)

    f1 = jnp.swapaxes(f1_t, 1, 2)
    stride = N // 256
    return _final_stage(f1, f_o[:, :, ::stride], w2, bn2, wf, bnf)
